# baseline (speedup 1.0000x reference)
import jax
import jax.numpy as jnp
from jax.experimental import pallas as pl
from jax.experimental.pallas import tpu as pltpu

_BM = 400


def _fused_kernel(feat_ref, w1_ref, w2_ref, a_ref, out_ref, xbuf, ybuf):
    p = pl.program_id(0)
    i = pl.program_id(1)

    @pl.when((p == 0) & (i == 0))
    def _prologue():
        w12 = jnp.dot(w1_ref[...], w2_ref[...], preferred_element_type=jnp.float32)
        g = jnp.dot(feat_ref[...], w12, preferred_element_type=jnp.float32)
        xbuf[...] = g.astype(jnp.bfloat16)

    @pl.when(p == 0)
    def _pass1():
        acc = jnp.dot(a_ref[...].astype(jnp.bfloat16), xbuf[...],
                      preferred_element_type=jnp.float32)
        ybuf[pl.ds(i * _BM, _BM), :] = acc.astype(jnp.bfloat16)

    @pl.when(p == 1)
    def _pass2():
        out_ref[...] = jnp.dot(a_ref[...].astype(jnp.bfloat16), ybuf[...],
                               preferred_element_type=jnp.float32)


@jax.jit
def kernel(feat, adj, W1, W2):
    n = adj.shape[0]
    f = W2.shape[1]
    return pl.pallas_call(
        _fused_kernel,
        grid=(2, n // _BM),
        in_specs=[
            pl.BlockSpec(feat.shape, lambda p, i: (0, 0)),
            pl.BlockSpec(W1.shape, lambda p, i: (0, 0)),
            pl.BlockSpec(W2.shape, lambda p, i: (0, 0)),
            pl.BlockSpec((_BM, n), lambda p, i: (i, 0)),
        ],
        out_specs=pl.BlockSpec((_BM, f), lambda p, i: (p * i, 0)),
        out_shape=jax.ShapeDtypeStruct((n, f), jnp.float32),
        scratch_shapes=[
            pltpu.VMEM((n, f), jnp.bfloat16),
            pltpu.VMEM((n, f), jnp.bfloat16),
        ],
    )(feat, W1, W2, adj)


# speedup vs baseline: 1.0175x; 1.0175x over previous
import jax
import jax.numpy as jnp
from jax.experimental import pallas as pl
from jax.experimental.pallas import tpu as pltpu

_BM = 400


def _fused_kernel(feat_ref, w1_ref, w2_ref, a_ref, out_ref, xbuf, ybuf):
    p = pl.program_id(0)
    i = pl.program_id(1)
    g = pl.num_programs(1)

    @pl.when((p == 0) & (i == 0))
    def _prologue():
        w12 = jnp.dot(w1_ref[...], w2_ref[...], preferred_element_type=jnp.float32)
        xbuf[...] = jnp.dot(feat_ref[...], w12, preferred_element_type=jnp.float32)

    @pl.when(p == 0)
    def _pass1():
        ybuf[pl.ds(i * _BM, _BM), :] = jnp.dot(
            a_ref[...], xbuf[...], preferred_element_type=jnp.float32)

    @pl.when(p == 1)
    def _pass2():
        out_ref[...] = jnp.dot(
            a_ref[...], ybuf[...], preferred_element_type=jnp.float32)


@jax.jit
def kernel(feat, adj, W1, W2):
    n = adj.shape[0]
    f = W2.shape[1]
    nblk = n // _BM
    return pl.pallas_call(
        _fused_kernel,
        grid=(2, nblk),
        in_specs=[
            pl.BlockSpec(feat.shape, lambda p, i: (0, 0)),
            pl.BlockSpec(W1.shape, lambda p, i: (0, 0)),
            pl.BlockSpec(W2.shape, lambda p, i: (0, 0)),
            pl.BlockSpec((_BM, n), lambda p, i: (i + p * (nblk - 1 - 2 * i), 0)),
        ],
        out_specs=pl.BlockSpec((_BM, f), lambda p, i: (nblk - 1 - p * i, 0)),
        out_shape=jax.ShapeDtypeStruct((n, f), jnp.float32),
        scratch_shapes=[
            pltpu.VMEM((n, f), jnp.float32),
            pltpu.VMEM((n, f), jnp.float32),
        ],
    )(feat, W1, W2, adj)


# speedup vs baseline: 1.0257x; 1.0080x over previous
import jax
import jax.numpy as jnp
from jax.experimental import pallas as pl
from jax.experimental.pallas import tpu as pltpu

_BM = 400


def _fused_kernel(feat_ref, w1_ref, w2_ref, a_ref, out_ref, xbuf, ybuf, abuf):
    p = pl.program_id(0)
    i = pl.program_id(1)
    nblk = pl.num_programs(1)

    @pl.when((p == 0) & (i == 0))
    def _prologue():
        w12 = jnp.dot(w1_ref[...], w2_ref[...], preferred_element_type=jnp.float32)
        g = jnp.dot(feat_ref[...], w12, preferred_element_type=jnp.float32)
        xbuf[...] = g.astype(jnp.bfloat16)
        abuf[...] = a_ref[...]

    @pl.when(p == 0)
    def _pass1():
        acc = jnp.dot(a_ref[...], xbuf[...].astype(jnp.float32),
                      preferred_element_type=jnp.float32)
        ybuf[pl.ds(i * _BM, _BM), :] = acc.astype(jnp.bfloat16)

    @pl.when((p == 1) & (i < nblk - 1))
    def _pass2():
        out_ref[...] = jnp.dot(a_ref[...], ybuf[...].astype(jnp.float32),
                               preferred_element_type=jnp.float32)

    @pl.when((p == 1) & (i == nblk - 1))
    def _pass2_cached():
        out_ref[...] = jnp.dot(abuf[...], ybuf[...].astype(jnp.float32),
                               preferred_element_type=jnp.float32)


@jax.jit
def kernel(feat, adj, W1, W2):
    n = adj.shape[0]
    f = W2.shape[1]
    nblk = n // _BM

    def a_idx(p, i):
        return (jnp.where(p == 0, i, jnp.maximum(nblk - 1 - i, 1)), 0)

    return pl.pallas_call(
        _fused_kernel,
        grid=(2, nblk),
        in_specs=[
            pl.BlockSpec(feat.shape, lambda p, i: (0, 0)),
            pl.BlockSpec(W1.shape, lambda p, i: (0, 0)),
            pl.BlockSpec(W2.shape, lambda p, i: (0, 0)),
            pl.BlockSpec((_BM, n), a_idx),
        ],
        out_specs=pl.BlockSpec((_BM, f), lambda p, i: (nblk - 1 - p * i, 0)),
        out_shape=jax.ShapeDtypeStruct((n, f), jnp.float32),
        scratch_shapes=[
            pltpu.VMEM((n, f), jnp.bfloat16),
            pltpu.VMEM((n, f), jnp.bfloat16),
            pltpu.VMEM((_BM, n), jnp.float32),
        ],
    )(feat, W1, W2, adj)


# speedup vs baseline: 1.0328x; 1.0068x over previous
import jax
import jax.numpy as jnp
from jax.experimental import pallas as pl
from jax.experimental.pallas import tpu as pltpu

_BM = 400


def _fused_kernel(feat_ref, w1_ref, w2_ref, a_ref, out_ref,
                  xbuf, ybuf, abuf0, abuf1):
    p = pl.program_id(0)
    i = pl.program_id(1)
    nblk = pl.num_programs(1)

    @pl.when((p == 0) & (i == 0))
    def _prologue():
        w12 = jnp.dot(w1_ref[...], w2_ref[...], preferred_element_type=jnp.float32)
        xbuf[...] = jnp.dot(feat_ref[...], w12, preferred_element_type=jnp.float32)
        abuf0[...] = a_ref[...].astype(jnp.bfloat16)

    @pl.when((p == 0) & (i == 1))
    def _cache1():
        abuf1[...] = a_ref[...].astype(jnp.bfloat16)

    @pl.when(p == 0)
    def _pass1():
        ybuf[pl.ds(i * _BM, _BM), :] = jnp.dot(
            a_ref[...], xbuf[...], preferred_element_type=jnp.float32)

    @pl.when((p == 1) & (i < nblk - 2))
    def _pass2():
        out_ref[...] = jnp.dot(
            a_ref[...], ybuf[...], preferred_element_type=jnp.float32)

    @pl.when((p == 1) & (i == nblk - 2))
    def _pass2_c1():
        out_ref[...] = jnp.dot(
            abuf1[...], ybuf[...].astype(jnp.bfloat16),
            preferred_element_type=jnp.float32)

    @pl.when((p == 1) & (i == nblk - 1))
    def _pass2_c0():
        out_ref[...] = jnp.dot(
            abuf0[...], ybuf[...].astype(jnp.bfloat16),
            preferred_element_type=jnp.float32)


@jax.jit
def kernel(feat, adj, W1, W2):
    n = adj.shape[0]
    f = W2.shape[1]
    nblk = n // _BM

    def a_idx(p, i):
        return (jnp.where(p == 0, i, jnp.maximum(nblk - 1 - i, 2)), 0)

    return pl.pallas_call(
        _fused_kernel,
        grid=(2, nblk),
        in_specs=[
            pl.BlockSpec(feat.shape, lambda p, i: (0, 0)),
            pl.BlockSpec(W1.shape, lambda p, i: (0, 0)),
            pl.BlockSpec(W2.shape, lambda p, i: (0, 0)),
            pl.BlockSpec((_BM, n), a_idx),
        ],
        out_specs=pl.BlockSpec((_BM, f), lambda p, i: (nblk - 1 - p * i, 0)),
        out_shape=jax.ShapeDtypeStruct((n, f), jnp.float32),
        scratch_shapes=[
            pltpu.VMEM((n, f), jnp.float32),
            pltpu.VMEM((n, f), jnp.float32),
            pltpu.VMEM((_BM, n), jnp.bfloat16),
            pltpu.VMEM((_BM, n), jnp.bfloat16),
        ],
        compiler_params=pltpu.CompilerParams(
            vmem_limit_bytes=64 * 1024 * 1024,
        ),
    )(feat, W1, W2, adj)
